# baseline (speedup 1.0000x reference)
import functools
import math

import jax
import jax.numpy as jnp
from jax import lax
from jax.experimental import pallas as pl
from jax.experimental.pallas import tpu as pltpu
from jax.experimental.pallas import tpu_sc as plsc

NC = 2
NS = 16
L = 16


def _pre_kernel(b1_ref, x_ref, gf_ref, w1a_ref, w1b_ref, t_ref, s_ref,
                *, G, Gp):
    i = pl.program_id(0)

    @pl.when(i == 0)
    def _s():
        s = lax.dot_general(jnp.maximum(gf_ref[...], 0.0), w1a_ref[...],
                            (((1,), (0,)), ((), ())),
                            preferred_element_type=jnp.float32)
        s_ref[0:G, :] = s + b1_ref[0, 0]
        s_ref[G:Gp, :] = jnp.zeros((Gp - G, 1), jnp.float32)

    t_ref[...] = lax.dot_general(x_ref[...], w1b_ref[...],
                                 (((1,), (0,)), ((), ())),
                                 preferred_element_type=jnp.float32)


def _pre(node_feats, g_feats, W1, b1, Gp, BT):
    V, F = node_feats.shape
    G = g_feats.shape[0]
    nb = V // BT
    return pl.pallas_call(
        functools.partial(_pre_kernel, G=G, Gp=Gp),
        grid=(nb,),
        in_specs=[
            pl.BlockSpec(memory_space=pltpu.SMEM),
            pl.BlockSpec((BT, F), lambda i: (i, 0)),
            pl.BlockSpec((G, F), lambda i: (0, 0)),
            pl.BlockSpec((F, 1), lambda i: (0, 0)),
            pl.BlockSpec((F, 1), lambda i: (0, 0)),
        ],
        out_specs=[
            pl.BlockSpec((BT, 1), lambda i: (i, 0)),
            pl.BlockSpec((Gp, 1), lambda i: (0, 0)),
        ],
        out_shape=[
            jax.ShapeDtypeStruct((V, 1), jnp.float32),
            jax.ShapeDtypeStruct((Gp, 1), jnp.float32),
        ],
    )(b1.reshape(1, 1), node_feats, g_feats, W1[:F], W1[F:])


def _sc_body(x_hbm, ids_hbm, t_hbm, s_hbm, u_out, d_out,
             ids_l, s_v, xb2, tb2, ezb, lxb, u_l, d_l,
             sx0, sx1, st0, st1, *, V, F, Gp, GW):
    cid = lax.axis_index("c")
    sid = lax.axis_index("s")
    w = cid * NS + sid
    gbase = w * GW
    KF = F // L
    iota = lax.iota(jnp.int32, L)

    pltpu.sync_copy(ids_hbm, ids_l)
    pltpu.sync_copy(s_hbm, s_v)

    def zrow(r, c):
        for k in range(KF):
            u_l[r, pl.ds(k * L, L)] = jnp.zeros((L,), jnp.float32)
        d_l[r, :] = jnp.zeros((L,), jnp.float32)
        return c
    lax.fori_loop(0, GW, zrow, 0)

    def lower_bound(thresh):
        lo = jnp.int32(0)
        p2 = 1
        while p2 * 2 <= V:
            p2 *= 2
        while p2 >= 1:
            cand = lo + p2
            probe = jnp.minimum(cand - 1, V - 1)
            pv = plsc.load_gather(ids_l, [jnp.full((L,), probe, jnp.int32)])
            ok = jnp.logical_and(cand <= V, jnp.max(pv) < thresh)
            lo = jnp.where(ok, lo + p2, lo)
            p2 //= 2
        return lo

    r0 = lower_bound(gbase)
    r1 = lower_bound(gbase + GW)
    p0 = jnp.minimum((r0 // 8) * 8, V - L)
    nbb = (r1 - p0 + L - 1) // L

    sems = ((sx0, st0), (sx1, st1))

    def issue(b, slot):
        p = jnp.minimum(p0 + b * L, V - L)
        pltpu.async_copy(x_hbm.at[pl.ds(p, L), :], xb2.at[slot],
                         sems[slot][0])
        pltpu.async_copy(t_hbm.at[pl.ds(p, L)], tb2.at[slot], sems[slot][1])

    def wait(slot):
        pltpu.make_async_copy(x_hbm.at[pl.ds(0, L), :], xb2.at[slot],
                              sems[slot][0]).wait()
        pltpu.make_async_copy(t_hbm.at[pl.ds(0, L)], tb2.at[slot],
                              sems[slot][1]).wait()

    @pl.when(nbb > 0)
    def _prime():
        issue(0, 0)

    def process(b, slot):
        p = jnp.minimum(p0 + b * L, V - L)
        absv = p + iota
        idv = plsc.load_gather(ids_l, [absv])
        tv = tb2[slot, :]
        active = jnp.logical_and(absv >= r0, absv < r1)
        sgv = plsc.load_gather(s_v, [jnp.where(active, idv, 0)])
        z = sgv + tv
        z = jnp.where(z >= 0.0, z, 0.01 * z)
        ez = jnp.where(active, jnp.exp(z), 0.0)
        lidx = jnp.where(active, idv - gbase, 0)
        ezb[...] = ez
        lxb[...] = lidx
        plsc.addupdate_scatter(d_l, [lidx, iota], ez)

        def jloop(j, c):
            jf = jnp.full((L,), j, jnp.int32)
            es = plsc.load_gather(ezb, [jf])
            lsp = plsc.load_gather(lxb, [jf])
            for k in range(KF):
                xk = xb2[slot, j, pl.ds(k * L, L)]
                plsc.addupdate_scatter(u_l, [lsp, k * L + iota], xk * es)
            return c
        lax.fori_loop(0, L, jloop, 0)

    def pair(pidx, c):
        for s_ in range(2):
            b = 2 * pidx + s_

            @pl.when(b < nbb)
            def _do():
                @pl.when(b + 1 < nbb)
                def _pf():
                    issue(b + 1, 1 - s_)
                wait(s_)
                process(b, s_)
        return c
    lax.fori_loop(0, (nbb + 1) // 2, pair, 0)

    pltpu.sync_copy(u_l, u_out.at[pl.ds(gbase, GW), :])
    pltpu.sync_copy(d_l, d_out.at[pl.ds(gbase, GW), :])


def _sc_stage(node_feats, graph_ids, t, s, Gp):
    V, F = node_feats.shape
    GW = Gp // (NC * NS)
    mesh = plsc.VectorSubcoreMesh(core_axis_name="c", subcore_axis_name="s")
    return pl.kernel(
        functools.partial(_sc_body, V=V, F=F, Gp=Gp, GW=GW),
        out_type=[
            jax.ShapeDtypeStruct((Gp, F), jnp.float32),
            jax.ShapeDtypeStruct((Gp, L), jnp.float32),
        ],
        mesh=mesh,
        compiler_params=pltpu.CompilerParams(needs_layout_passes=False),
        scratch_types=[
            pltpu.VMEM((V,), jnp.int32),
            pltpu.VMEM((Gp,), jnp.float32),
            pltpu.VMEM((2, L, F), jnp.float32),
            pltpu.VMEM((2, L), jnp.float32),
            pltpu.VMEM((L,), jnp.float32),
            pltpu.VMEM((L,), jnp.int32),
            pltpu.VMEM((GW, F), jnp.float32),
            pltpu.VMEM((GW, L), jnp.float32),
            pltpu.SemaphoreType.DMA,
            pltpu.SemaphoreType.DMA,
            pltpu.SemaphoreType.DMA,
            pltpu.SemaphoreType.DMA,
        ],
    )(node_feats, graph_ids, t, s)


def _final_kernel(u_ref, d2_ref, w2_ref, b2_ref, out_ref, *, G):
    d = jnp.sum(d2_ref[...], axis=1, keepdims=True)
    dsafe = jnp.where(d > 0, d, 1.0)
    S = u_ref[...] / dsafe
    rep = lax.dot_general(S, w2_ref[...], (((1,), (0,)), ((), ())),
                          preferred_element_type=jnp.float32)
    rep = rep + jnp.where(d > 0, 1.0, 0.0) * b2_ref[...]
    out_ref[...] = rep[0:G, :]


def _final(u, d2, W2, b2, G):
    Gp, F = u.shape
    return pl.pallas_call(
        functools.partial(_final_kernel, G=G),
        in_specs=[
            pl.BlockSpec((Gp, F), lambda: (0, 0)),
            pl.BlockSpec((Gp, L), lambda: (0, 0)),
            pl.BlockSpec((F, F), lambda: (0, 0)),
            pl.BlockSpec((1, F), lambda: (0, 0)),
        ],
        out_specs=pl.BlockSpec((G, F), lambda: (0, 0)),
        out_shape=jax.ShapeDtypeStruct((G, F), jnp.float32),
    )(u, d2, W2, b2.reshape(1, F))


@jax.jit
def kernel(node_feats, g_feats, graph_ids, W1, b1, W2, b2):
    V, F = node_feats.shape
    G = g_feats.shape[0]
    NW = NC * NS
    Gp = math.ceil(G / (NW * 8)) * (NW * 8)
    BT = next(b for b in (1000, 512, 500, 400, 256, 250, 200, 128, 125,
                          100, 64, 50, 40, 32, 25, 16, 8, V) if V % b == 0)

    t, s = _pre(node_feats, g_feats, W1, b1, Gp, BT)
    u, d2 = _sc_stage(node_feats, graph_ids.astype(jnp.int32),
                      t.reshape(V), s.reshape(Gp), Gp)
    return _final(u, d2, W2, b2, G)


# speedup vs baseline: 1.2830x; 1.2830x over previous
import functools
import math

import jax
import jax.numpy as jnp
from jax import lax
from jax.experimental import pallas as pl
from jax.experimental.pallas import tpu as pltpu
from jax.experimental.pallas import tpu_sc as plsc

NC = 2
NS = 16
L = 16


def _pre_kernel(b1_ref, x_ref, gf_ref, w1a_ref, w1b_ref, t_ref, s_ref,
                *, G, Gp):
    i = pl.program_id(0)

    @pl.when(i == 0)
    def _s():
        s = lax.dot_general(jnp.maximum(gf_ref[...], 0.0), w1a_ref[...],
                            (((1,), (0,)), ((), ())),
                            preferred_element_type=jnp.float32)
        s_ref[0:G, :] = s + b1_ref[0, 0]
        s_ref[G:Gp, :] = jnp.zeros((Gp - G, 1), jnp.float32)

    t_ref[...] = lax.dot_general(x_ref[...], w1b_ref[...],
                                 (((1,), (0,)), ((), ())),
                                 preferred_element_type=jnp.float32)


def _pre(node_feats, g_feats, W1, b1, Gp, BT):
    V, F = node_feats.shape
    G = g_feats.shape[0]
    nb = V // BT
    return pl.pallas_call(
        functools.partial(_pre_kernel, G=G, Gp=Gp),
        grid=(nb,),
        in_specs=[
            pl.BlockSpec(memory_space=pltpu.SMEM),
            pl.BlockSpec((BT, F), lambda i: (i, 0)),
            pl.BlockSpec((G, F), lambda i: (0, 0)),
            pl.BlockSpec((F, 1), lambda i: (0, 0)),
            pl.BlockSpec((F, 1), lambda i: (0, 0)),
        ],
        out_specs=[
            pl.BlockSpec((BT, 1), lambda i: (i, 0)),
            pl.BlockSpec((Gp, 1), lambda i: (0, 0)),
        ],
        out_shape=[
            jax.ShapeDtypeStruct((V, 1), jnp.float32),
            jax.ShapeDtypeStruct((Gp, 1), jnp.float32),
        ],
    )(b1.reshape(1, 1), node_feats, g_feats, W1[:F], W1[F:])


def _sc_body(x_hbm, ids_hbm, t_hbm, s_hbm, u_out, d_out,
             ids_l, tb, s_v, xb, ezb, u_l, d_l, *, V, F, Gp, GW):
    cid = lax.axis_index("c")
    sid = lax.axis_index("s")
    w = cid * NS + sid
    gbase = w * GW
    KF = F // L
    NG = 3
    NB = NG * L
    iota = lax.iota(jnp.int32, L)

    pltpu.sync_copy(ids_hbm, ids_l)
    pltpu.sync_copy(s_hbm, s_v)

    lo = jnp.int32(0)
    p2 = 1
    while p2 * 2 <= V:
        p2 *= 2
    while p2 >= 1:
        cand = lo + p2
        probe = jnp.minimum(cand - 1, V - 1)
        pv = plsc.load_gather(ids_l, [jnp.full((L,), probe, jnp.int32)])
        ok = jnp.logical_and(cand <= V, jnp.max(pv) < gbase)
        lo = jnp.where(ok, lo + p2, lo)
        p2 //= 2

    def gloop(gl, ptr):
        gg = gbase + gl
        sgv = plsc.load_gather(s_v, [jnp.full((L,), gg, jnp.int32)])
        zero = jnp.zeros((L,), jnp.float32)

        def cond(st):
            return st[0]

        def body(st):
            ptr_ = st[1]
            dacc = st[2]
            accs = st[3:]
            p = jnp.minimum((ptr_ // 8) * 8, V - NB)
            pltpu.sync_copy(x_hbm.at[pl.ds(p, NB), :], xb)
            pltpu.sync_copy(t_hbm.at[pl.ds(p, NB)], tb)
            nm = jnp.int32(0)
            dadd = jnp.zeros((L,), jnp.float32)
            for g in range(NG):
                absg = p + g * L + iota
                idg = plsc.load_gather(ids_l, [absg])
                tvg = tb[pl.ds(g * L, L)]
                maskg = jnp.logical_and(idg == gg, absg >= ptr_)
                zg = sgv + tvg
                zg = jnp.where(zg >= 0.0, zg, 0.01 * zg)
                ezg = jnp.where(maskg, jnp.exp(zg), 0.0)
                ezb[pl.ds(g * L, L)] = ezg
                dadd = dadd + ezg
                nm = nm + jnp.sum(jnp.where(maskg, 1, 0))

            def jloop(j, a):
                es = plsc.load_gather(ezb, [jnp.full((L,), j, jnp.int32)])
                return tuple(a[k] + xb[j, pl.ds(k * L, L)] * es
                             for k in range(KF))
            new_accs = lax.fori_loop(0, NB, jloop, tuple(accs))
            avail = p + NB - ptr_
            go = jnp.logical_and(nm == avail, avail > 0)
            return (go, ptr_ + nm, dacc + dadd) + tuple(new_accs)

        init = (jnp.bool_(True), ptr, zero) + (zero,) * KF
        st = lax.while_loop(cond, body, init)
        for k in range(KF):
            u_l[gl, pl.ds(k * L, L)] = st[3 + k]
        d_l[gl, :] = st[2]
        return st[1]

    lax.fori_loop(0, GW, gloop, lo)
    pltpu.sync_copy(u_l, u_out.at[pl.ds(gbase, GW), :])
    pltpu.sync_copy(d_l, d_out.at[pl.ds(gbase, GW), :])


def _sc_stage(node_feats, graph_ids, t, s, Gp):
    V, F = node_feats.shape
    GW = Gp // (NC * NS)
    mesh = plsc.VectorSubcoreMesh(core_axis_name="c", subcore_axis_name="s")
    return pl.kernel(
        functools.partial(_sc_body, V=V, F=F, Gp=Gp, GW=GW),
        out_type=[
            jax.ShapeDtypeStruct((Gp, F), jnp.float32),
            jax.ShapeDtypeStruct((Gp, L), jnp.float32),
        ],
        mesh=mesh,
        compiler_params=pltpu.CompilerParams(needs_layout_passes=False),
        scratch_types=[
            pltpu.VMEM((V,), jnp.int32),
            pltpu.VMEM((48,), jnp.float32),
            pltpu.VMEM((Gp,), jnp.float32),
            pltpu.VMEM((48, F), jnp.float32),
            pltpu.VMEM((48,), jnp.float32),
            pltpu.VMEM((GW, F), jnp.float32),
            pltpu.VMEM((GW, L), jnp.float32),
        ],
    )(node_feats, graph_ids, t, s)


def _final_kernel(u_ref, d2_ref, w2_ref, b2_ref, out_ref, *, G):
    d = jnp.sum(d2_ref[...], axis=1, keepdims=True)
    dsafe = jnp.where(d > 0, d, 1.0)
    S = u_ref[...] / dsafe
    rep = lax.dot_general(S, w2_ref[...], (((1,), (0,)), ((), ())),
                          preferred_element_type=jnp.float32)
    rep = rep + jnp.where(d > 0, 1.0, 0.0) * b2_ref[...]
    out_ref[...] = rep[0:G, :]


def _final(u, d2, W2, b2, G):
    Gp, F = u.shape
    return pl.pallas_call(
        functools.partial(_final_kernel, G=G),
        in_specs=[
            pl.BlockSpec((Gp, F), lambda: (0, 0)),
            pl.BlockSpec((Gp, L), lambda: (0, 0)),
            pl.BlockSpec((F, F), lambda: (0, 0)),
            pl.BlockSpec((1, F), lambda: (0, 0)),
        ],
        out_specs=pl.BlockSpec((G, F), lambda: (0, 0)),
        out_shape=jax.ShapeDtypeStruct((G, F), jnp.float32),
    )(u, d2, W2, b2.reshape(1, F))


@jax.jit
def kernel(node_feats, g_feats, graph_ids, W1, b1, W2, b2):
    V, F = node_feats.shape
    G = g_feats.shape[0]
    NW = NC * NS
    Gp = math.ceil(G / (NW * 8)) * (NW * 8)
    BT = next(b for b in (1000, 512, 500, 400, 256, 250, 200, 128, 125,
                          100, 64, 50, 40, 32, 25, 16, 8, V) if V % b == 0)

    t, s = _pre(node_feats, g_feats, W1, b1, Gp, BT)
    u, d2 = _sc_stage(node_feats, graph_ids.astype(jnp.int32),
                      t.reshape(V), s.reshape(Gp), Gp)
    return _final(u, d2, W2, b2, G)


# speedup vs baseline: 1.3774x; 1.0736x over previous
import functools
import math

import jax
import jax.numpy as jnp
from jax import lax
from jax.experimental import pallas as pl
from jax.experimental.pallas import tpu as pltpu
from jax.experimental.pallas import tpu_sc as plsc

NC = 2
NS = 16
L = 16


def _pre_kernel(b1_ref, x_ref, gf_ref, w1a_ref, w1b_ref, t_ref, s_ref,
                *, G, Gp):
    i = pl.program_id(0)

    @pl.when(i == 0)
    def _s():
        s = lax.dot_general(jnp.maximum(gf_ref[...], 0.0), w1a_ref[...],
                            (((1,), (0,)), ((), ())),
                            preferred_element_type=jnp.float32)
        s_ref[0:G, :] = s + b1_ref[0, 0]
        s_ref[G:Gp, :] = jnp.zeros((Gp - G, 1), jnp.float32)

    t_ref[...] = lax.dot_general(x_ref[...], w1b_ref[...],
                                 (((1,), (0,)), ((), ())),
                                 preferred_element_type=jnp.float32)


def _pre(node_feats, g_feats, W1, b1, Gp, BT):
    V, F = node_feats.shape
    G = g_feats.shape[0]
    nb = V // BT
    return pl.pallas_call(
        functools.partial(_pre_kernel, G=G, Gp=Gp),
        grid=(nb,),
        in_specs=[
            pl.BlockSpec(memory_space=pltpu.SMEM),
            pl.BlockSpec((BT, F), lambda i: (i, 0)),
            pl.BlockSpec((G, F), lambda i: (0, 0)),
            pl.BlockSpec((F, 1), lambda i: (0, 0)),
            pl.BlockSpec((F, 1), lambda i: (0, 0)),
        ],
        out_specs=[
            pl.BlockSpec((BT, 1), lambda i: (i, 0)),
            pl.BlockSpec((Gp, 1), lambda i: (0, 0)),
        ],
        out_shape=[
            jax.ShapeDtypeStruct((V, 1), jnp.float32),
            jax.ShapeDtypeStruct((Gp, 1), jnp.float32),
        ],
    )(b1.reshape(1, 1), node_feats, g_feats, W1[:F], W1[F:])


def _sc_body(x_hbm, ids_hbm, t_hbm, s_hbm, u_out, d_out,
             ids_l, tb, s_v, xb, ezb, u_l, d_l, *, V, F, Gp, GW):
    cid = lax.axis_index("c")
    sid = lax.axis_index("s")
    w = cid * NS + sid
    gbase = w * GW
    KF = F // L
    NG = 6
    NB = NG * L
    iota = lax.iota(jnp.int32, L)

    pltpu.sync_copy(ids_hbm, ids_l)
    pltpu.sync_copy(s_hbm, s_v)

    lo = jnp.int32(0)
    p2 = 1
    while p2 * 2 <= V:
        p2 *= 2
    while p2 >= 1:
        cand = lo + p2
        probe = jnp.minimum(cand - 1, V - 1)
        pv = plsc.load_gather(ids_l, [jnp.full((L,), probe, jnp.int32)])
        ok = jnp.logical_and(cand <= V, jnp.max(pv) < gbase)
        lo = jnp.where(ok, lo + p2, lo)
        p2 //= 2

    def gloop(gl, ptr):
        gg = gbase + gl
        sgv = plsc.load_gather(s_v, [jnp.full((L,), gg, jnp.int32)])
        zero = jnp.zeros((L,), jnp.float32)

        def cond(st):
            return st[0]

        def body(st):
            ptr_ = st[1]
            dacc = st[2]
            accs = st[3:]
            p = jnp.minimum((ptr_ // 8) * 8, V - NB)
            pltpu.sync_copy(x_hbm.at[pl.ds(p, NB), :], xb)
            pltpu.sync_copy(t_hbm.at[pl.ds(p, NB)], tb)
            nm = jnp.int32(0)
            dadd = jnp.zeros((L,), jnp.float32)
            for g in range(NG):
                absg = p + g * L + iota
                idg = plsc.load_gather(ids_l, [absg])
                tvg = tb[pl.ds(g * L, L)]
                maskg = jnp.logical_and(idg == gg, absg >= ptr_)
                zg = sgv + tvg
                zg = jnp.where(zg >= 0.0, zg, 0.01 * zg)
                ezg = jnp.where(maskg, jnp.exp(zg), 0.0)
                ezb[pl.ds(g * L, L)] = ezg
                dadd = dadd + ezg
                nm = nm + jnp.sum(jnp.where(maskg, 1, 0))

            def jloop(j, a):
                es = plsc.load_gather(ezb, [jnp.full((L,), j, jnp.int32)])
                return tuple(a[k] + xb[j, pl.ds(k * L, L)] * es
                             for k in range(KF))
            new_accs = lax.fori_loop(0, NB, jloop, tuple(accs))
            avail = p + NB - ptr_
            go = jnp.logical_and(nm == avail, avail > 0)
            return (go, ptr_ + nm, dacc + dadd) + tuple(new_accs)

        init = (jnp.bool_(True), ptr, zero) + (zero,) * KF
        st = lax.while_loop(cond, body, init)
        for k in range(KF):
            u_l[gl, pl.ds(k * L, L)] = st[3 + k]
        d_l[gl, :] = st[2]
        return st[1]

    lax.fori_loop(0, GW, gloop, lo)
    pltpu.sync_copy(u_l, u_out.at[pl.ds(gbase, GW), :])
    pltpu.sync_copy(d_l, d_out.at[pl.ds(gbase, GW), :])


def _sc_stage(node_feats, graph_ids, t, s, Gp):
    V, F = node_feats.shape
    GW = Gp // (NC * NS)
    mesh = plsc.VectorSubcoreMesh(core_axis_name="c", subcore_axis_name="s")
    return pl.kernel(
        functools.partial(_sc_body, V=V, F=F, Gp=Gp, GW=GW),
        out_type=[
            jax.ShapeDtypeStruct((Gp, F), jnp.float32),
            jax.ShapeDtypeStruct((Gp, L), jnp.float32),
        ],
        mesh=mesh,
        compiler_params=pltpu.CompilerParams(needs_layout_passes=False),
        scratch_types=[
            pltpu.VMEM((V,), jnp.int32),
            pltpu.VMEM((96,), jnp.float32),
            pltpu.VMEM((Gp,), jnp.float32),
            pltpu.VMEM((96, F), jnp.float32),
            pltpu.VMEM((96,), jnp.float32),
            pltpu.VMEM((GW, F), jnp.float32),
            pltpu.VMEM((GW, L), jnp.float32),
        ],
    )(node_feats, graph_ids, t, s)


def _final_kernel(u_ref, d2_ref, w2_ref, b2_ref, out_ref, *, G):
    d = jnp.sum(d2_ref[...], axis=1, keepdims=True)
    dsafe = jnp.where(d > 0, d, 1.0)
    S = u_ref[...] / dsafe
    rep = lax.dot_general(S, w2_ref[...], (((1,), (0,)), ((), ())),
                          preferred_element_type=jnp.float32)
    rep = rep + jnp.where(d > 0, 1.0, 0.0) * b2_ref[...]
    out_ref[...] = rep[0:G, :]


def _final(u, d2, W2, b2, G):
    Gp, F = u.shape
    return pl.pallas_call(
        functools.partial(_final_kernel, G=G),
        in_specs=[
            pl.BlockSpec((Gp, F), lambda: (0, 0)),
            pl.BlockSpec((Gp, L), lambda: (0, 0)),
            pl.BlockSpec((F, F), lambda: (0, 0)),
            pl.BlockSpec((1, F), lambda: (0, 0)),
        ],
        out_specs=pl.BlockSpec((G, F), lambda: (0, 0)),
        out_shape=jax.ShapeDtypeStruct((G, F), jnp.float32),
    )(u, d2, W2, b2.reshape(1, F))


@jax.jit
def kernel(node_feats, g_feats, graph_ids, W1, b1, W2, b2):
    V, F = node_feats.shape
    G = g_feats.shape[0]
    NW = NC * NS
    Gp = math.ceil(G / (NW * 8)) * (NW * 8)
    BT = next(b for b in (1000, 512, 500, 400, 256, 250, 200, 128, 125,
                          100, 64, 50, 40, 32, 25, 16, 8, V) if V % b == 0)

    t, s = _pre(node_feats, g_feats, W1, b1, Gp, BT)
    u, d2 = _sc_stage(node_feats, graph_ids.astype(jnp.int32),
                      t.reshape(V), s.reshape(Gp), Gp)
    return _final(u, d2, W2, b2, G)
